# baseline (speedup 1.0000x reference)
import functools

import jax
import jax.numpy as jnp
from jax import lax
from jax.experimental import pallas as pl
from jax.experimental.pallas import tpu as pltpu
from jax.experimental.pallas import tpu_sc as plsc

N = 10000
E = 160000
F = 128
FP = 144
NREP = 4
KE = 128
CROWS = 1280
EPAD = CROWS * KE
RPT = CROWS // 16
NBUF = 2
NB = RPT // NBUF
ROWS = 624


def _sc_aggregate(src2, dst2, xaug3, zrows):
  mesh = plsc.VectorSubcoreMesh(core_axis_name="c", subcore_axis_name="s")

  @functools.partial(
      pl.kernel,
      mesh=mesh,
      compiler_params=pltpu.CompilerParams(use_tc_tiling_on_sc=False),
      out_type=jax.ShapeDtypeStruct((NREP * N, FP), jnp.float32),
      scratch_types=[
          pltpu.VMEM((2, NBUF, KE), jnp.int32),
          pltpu.VMEM((2, NBUF, KE), jnp.int32),
          pltpu.VMEM((NBUF, KE, FP), jnp.float32),
          pltpu.VMEM_SHARED((N, FP), jnp.float32),
          pltpu.SemaphoreType.DMA((NBUF,)),
          pltpu.SemaphoreType.DMA((NBUF,)),
          pltpu.SemaphoreType.DMA((2,)),
      ],
  )
  def k(src2_hbm, dst2_hbm, xaug_hbm, z_hbm, out_hbm,
        si_v, di_v, rows_v, acc, gsem, ssem, isem):
    c = lax.axis_index("c")
    s = lax.axis_index("s")
    row_base = s * ROWS
    crow0 = s * RPT

    def idxblock(m, w):
      sl = pl.ds(crow0 + m * NBUF, NBUF)
      pltpu.async_copy(src2_hbm.at[sl], si_v.at[w], isem.at[w])
      pltpu.async_copy(dst2_hbm.at[sl], di_v.at[w], isem.at[w])

    def iwait(w):
      pltpu.make_async_copy(
          src2_hbm.at[pl.ds(0, NBUF)], si_v.at[w], isem.at[w]).wait()
      pltpu.make_async_copy(
          dst2_hbm.at[pl.ds(0, NBUF)], di_v.at[w], isem.at[w]).wait()

    def gwait(u):
      pltpu.make_async_copy(
          xaug_hbm.at[0].at[pl.ds(0, KE)], rows_v.at[u], gsem.at[u]).wait()

    def swait(u):
      pltpu.make_async_copy(
          xaug_hbm.at[0].at[pl.ds(0, KE)], rows_v.at[u], ssem.at[u]).wait()

    def sstart(w, u):
      pltpu.async_copy(rows_v.at[u], acc.at[di_v.at[w].at[u]],
                       ssem.at[u], add=True)

    for rr in range(NREP // 2):
      r = rr * 2 + c

      def gstart(w, u):
        pltpu.async_copy(xaug_hbm.at[r].at[si_v.at[w].at[u]],
                         rows_v.at[u], gsem.at[u])

      pltpu.sync_copy(z_hbm.at[pl.ds(0, ROWS)],
                      acc.at[pl.ds(row_base, ROWS)])

      @pl.when(s == 15)
      def _():
        pltpu.sync_copy(z_hbm.at[pl.ds(0, 16)], acc.at[pl.ds(16 * ROWS, 16)])

      plsc.subcore_barrier()

      idxblock(0, 0)
      iwait(0)
      for u in range(NBUF):
        gstart(0, u)
      idxblock(1, 1)

      def pair(ii, carry):
        for w in range(2):
          i = ii * 2 + w
          for u in range(NBUF):
            gwait(u)

          @pl.when(i + 1 < NB)
          def _():
            iwait(1 - w)
            for u in range(NBUF):
              gstart(1 - w, u)

            @pl.when(i + 2 < NB)
            def _():
              idxblock(i + 2, w)
        return carry

      lax.fori_loop(0, NB // 2, pair, 0)

      plsc.subcore_barrier()

      pltpu.sync_copy(acc.at[pl.ds(row_base, ROWS)],
                      out_hbm.at[pl.ds(r * N + row_base, ROWS)])

      @pl.when(s == 15)
      def _():
        pltpu.sync_copy(acc.at[pl.ds(16 * ROWS, 16)],
                        out_hbm.at[pl.ds(r * N + 16 * ROWS, 16)])

      plsc.subcore_barrier()

  return k(src2, dst2, xaug3, zrows)


def _tc_body(x_ref, agg_ref, wn_ref, ws_ref, b_ref, o_ref):
  wn = wn_ref[...]
  ws = ws_ref[...]
  bias = b_ref[0, :]
  for t in range(2):
    x = x_ref[0, :, t, :]
    ag = agg_ref[t, :, 0:F]
    deg = agg_ref[t, :, F:F + 1]
    mean = ag / jnp.maximum(deg, 1.0)
    acc = jnp.dot(mean, wn, preferred_element_type=jnp.float32,
                  precision=lax.Precision.HIGHEST)
    acc += jnp.dot(x, ws, preferred_element_type=jnp.float32,
                   precision=lax.Precision.HIGHEST)
    o_ref[0, :, t, :] = jnp.maximum(acc + bias, 0.0)


def kernel(X, g, W_self, W_neigh, b):
  B, n, T, f = X.shape
  src = g[0]
  dst = g[1]

  t1 = jnp.transpose(X, (0, 2, 1, 3)).reshape(NREP * N, F)
  xaug = jnp.concatenate(
      [t1,
       jnp.ones((NREP * N, 1), jnp.float32),
       jnp.zeros((NREP * N, FP - F - 1), jnp.float32)], axis=1)
  xaug3 = jnp.concatenate(
      [xaug.reshape(NREP, N, FP), jnp.zeros((NREP, 8, FP), jnp.float32)],
      axis=1)
  zrows = jnp.zeros((ROWS, FP), jnp.float32)

  pad = EPAD - E
  src_p = jnp.concatenate(
      [src, jnp.full((pad,), N, dtype=jnp.int32)])
  dst_p = jnp.concatenate(
      [dst, jnp.arange(pad, dtype=jnp.int32) % N])
  src2 = src_p.reshape(CROWS, KE)
  dst2 = dst_p.reshape(CROWS, KE)

  agg = _sc_aggregate(src2, dst2, xaug3, zrows).reshape(NREP, N, FP)

  RB = 2000
  out = pl.pallas_call(
      _tc_body,
      grid=(B, N // RB),
      in_specs=[
          pl.BlockSpec((1, RB, T, F), lambda bb, nn: (bb, nn, 0, 0)),
          pl.BlockSpec((T, RB, FP), lambda bb, nn: (bb, nn, 0)),
          pl.BlockSpec((F, F), lambda bb, nn: (0, 0)),
          pl.BlockSpec((F, F), lambda bb, nn: (0, 0)),
          pl.BlockSpec((1, F), lambda bb, nn: (0, 0)),
      ],
      out_specs=pl.BlockSpec((1, RB, T, F), lambda bb, nn: (bb, nn, 0, 0)),
      out_shape=jax.ShapeDtypeStruct((B, N, T, F), jnp.float32),
  )(X, agg, W_neigh.T, W_self.T, b.reshape(1, F))
  return out


# speedup vs baseline: 1.1983x; 1.1983x over previous
import functools

import jax
import jax.numpy as jnp
from jax import lax
from jax.experimental import pallas as pl
from jax.experimental.pallas import tpu as pltpu
from jax.experimental.pallas import tpu_sc as plsc

N = 10000
E = 160000
F = 128
FP = 144
NREP = 4
KE = 128
NCHUNKS = E // KE
ROWS = 624


def _sc_aggregate(xaug, src, dst, zrows):
  mesh = plsc.VectorSubcoreMesh(core_axis_name="c", subcore_axis_name="s")

  @functools.partial(
      pl.kernel,
      mesh=mesh,
      compiler_params=pltpu.CompilerParams(use_tc_tiling_on_sc=False),
      out_type=jax.ShapeDtypeStruct((NREP * N, FP), jnp.float32),
      scratch_types=[
          pltpu.VMEM((KE,), jnp.int32),
          pltpu.VMEM((KE,), jnp.int32),
          pltpu.VMEM((KE,), jnp.int32),
          pltpu.VMEM((KE, FP), jnp.float32),
          pltpu.VMEM_SHARED((N, FP), jnp.float32),
          pltpu.SemaphoreType.DMA,
      ],
  )
  def k(xaug_hbm, src_hbm, dst_hbm, z_hbm, out_hbm,
        si_v, so_v, di_v, rows_v, acc, sem):
    c = lax.axis_index("c")
    s = lax.axis_index("s")
    row_base = s * ROWS
    nch = jnp.where(s < NCHUNKS - (NCHUNKS // 16) * 16, NCHUNKS // 16 + 1,
                    NCHUNKS // 16)

    for rr in range(NREP // 2):
      r = rr * 2 + c
      roff = r * N

      pltpu.sync_copy(z_hbm.at[pl.ds(0, ROWS)],
                      acc.at[pl.ds(row_base, ROWS)])

      @pl.when(s == 15)
      def _():
        pltpu.sync_copy(z_hbm.at[pl.ds(0, 16)], acc.at[pl.ds(16 * ROWS, 16)])

      plsc.subcore_barrier()

      def chunk(i, carry):
        e0 = (i * 16 + s) * KE
        pltpu.sync_copy(src_hbm.at[pl.ds(e0, KE)], si_v)
        pltpu.sync_copy(dst_hbm.at[pl.ds(e0, KE)], di_v)
        for j in range(KE // 16):
          sl = pl.ds(j * 16, 16)
          so_v[sl] = si_v[sl] + roff
        pltpu.async_copy(xaug_hbm.at[so_v], rows_v, sem).wait()
        pltpu.sync_copy(rows_v, acc.at[di_v], add=True)
        return carry

      lax.fori_loop(0, nch, chunk, 0)

      plsc.subcore_barrier()

      pltpu.sync_copy(acc.at[pl.ds(row_base, ROWS)],
                      out_hbm.at[pl.ds(roff + row_base, ROWS)])

      @pl.when(s == 15)
      def _():
        pltpu.sync_copy(acc.at[pl.ds(16 * ROWS, 16)],
                        out_hbm.at[pl.ds(roff + 16 * ROWS, 16)])

      plsc.subcore_barrier()

  return k(xaug, src, dst, zrows)


def _tc_body(x_ref, agg_ref, wn_ref, ws_ref, b_ref, o_ref):
  wn = wn_ref[...]
  ws = ws_ref[...]
  bias = b_ref[0, :]
  for t in range(2):
    x = x_ref[0, :, t, :]
    ag = agg_ref[t, :, 0:F]
    deg = agg_ref[t, :, F:F + 1]
    mean = ag / jnp.maximum(deg, 1.0)
    acc = jnp.dot(mean, wn, preferred_element_type=jnp.float32,
                  precision=lax.Precision.HIGHEST)
    acc += jnp.dot(x, ws, preferred_element_type=jnp.float32,
                   precision=lax.Precision.HIGHEST)
    o_ref[0, :, t, :] = jnp.maximum(acc + bias, 0.0)


def kernel(X, g, W_self, W_neigh, b):
  B, n, T, f = X.shape
  src = g[0]
  dst = g[1]

  t1 = jnp.transpose(X, (0, 2, 1, 3)).reshape(NREP * N, F)
  xaug = jnp.concatenate(
      [t1,
       jnp.ones((NREP * N, 1), jnp.float32),
       jnp.zeros((NREP * N, FP - F - 1), jnp.float32)], axis=1)
  zrows = jnp.zeros((ROWS, FP), jnp.float32)

  agg = _sc_aggregate(xaug, src, dst, zrows).reshape(NREP, N, FP)

  RB = 2000
  out = pl.pallas_call(
      _tc_body,
      grid=(B, N // RB),
      in_specs=[
          pl.BlockSpec((1, RB, T, F), lambda bb, nn: (bb, nn, 0, 0)),
          pl.BlockSpec((T, RB, FP), lambda bb, nn: (bb, nn, 0)),
          pl.BlockSpec((F, F), lambda bb, nn: (0, 0)),
          pl.BlockSpec((F, F), lambda bb, nn: (0, 0)),
          pl.BlockSpec((1, F), lambda bb, nn: (0, 0)),
      ],
      out_specs=pl.BlockSpec((1, RB, T, F), lambda bb, nn: (bb, nn, 0, 0)),
      out_shape=jax.ShapeDtypeStruct((B, N, T, F), jnp.float32),
  )(X, agg, W_neigh.T, W_self.T, b.reshape(1, F))
  return out


# speedup vs baseline: 1.8731x; 1.5631x over previous
import functools

import jax
import jax.numpy as jnp
from jax import lax
from jax.experimental import pallas as pl
from jax.experimental.pallas import tpu as pltpu
from jax.experimental.pallas import tpu_sc as plsc

N = 10000
E = 160000
F = 128
FP = 144
NREP = 4
KE = 128
CROWS = 1280
EPAD = CROWS * KE
RPT = CROWS // 16
NBUF = 2
ZR = 128
ROWS = 624


def _sc_aggregate(src2, dst2, xaug3, zrows):
  mesh = plsc.VectorSubcoreMesh(core_axis_name="c", subcore_axis_name="s")

  @functools.partial(
      pl.kernel,
      mesh=mesh,
      compiler_params=pltpu.CompilerParams(use_tc_tiling_on_sc=False),
      out_type=jax.ShapeDtypeStruct((NREP * N, FP), jnp.float32),
      scratch_types=(
          [pltpu.VMEM((KE,), jnp.int32)] * 4 +
          [pltpu.VMEM((KE,), jnp.int32)] * 4 +
          [pltpu.VMEM((NBUF, KE, FP), jnp.float32),
           pltpu.VMEM_SHARED((N, FP), jnp.float32),
           pltpu.SemaphoreType.DMA((NBUF,)),
           pltpu.SemaphoreType.DMA((4,))]
      ),
  )
  def k(src2_hbm, dst2_hbm, xaug_hbm, z_hbm, out_hbm,
        si0, si1, si2, si3, di0, di1, di2, di3, rows_v, acc, gsem, isem):
    sis = [si0, si1, si2, si3]
    dis = [di0, di1, di2, di3]
    c = lax.axis_index("c")
    s = lax.axis_index("s")
    row_base = s * ROWS
    cbase = s * RPT

    def idxload(j, q):
      pltpu.async_copy(src2_hbm.at[cbase + j], sis[q], isem.at[q])
      pltpu.async_copy(dst2_hbm.at[cbase + j], dis[q], isem.at[q])

    def iwait(q):
      pltpu.make_async_copy(src2_hbm.at[0], sis[q], isem.at[q]).wait()
      pltpu.make_async_copy(dst2_hbm.at[0], dis[q], isem.at[q]).wait()

    def gwait(p):
      pltpu.make_async_copy(
          xaug_hbm.at[0].at[pl.ds(0, KE)], rows_v.at[p], gsem.at[p]).wait()

    def scatter(q, p):
      pltpu.sync_copy(rows_v.at[p], acc.at[dis[q]], add=True)

    for rr in range(NREP // 2):
      r = rr * 2 + c

      def gstart(q, p):
        pltpu.async_copy(
            xaug_hbm.at[r].at[sis[q]], rows_v.at[p], gsem.at[p])

      pltpu.sync_copy(z_hbm.at[pl.ds(0, ROWS)],
                      acc.at[pl.ds(row_base, ROWS)])

      @pl.when(s == 15)
      def _():
        pltpu.sync_copy(z_hbm.at[pl.ds(0, 16)], acc.at[pl.ds(16 * ROWS, 16)])

      plsc.subcore_barrier()

      idxload(0, 0)
      idxload(1, 1)
      iwait(0)
      gstart(0, 0)
      iwait(1)
      gstart(1, 1)

      def blk(i, carry):
        for pp in range(4):
          j = i * 4 + pp
          p = pp % NBUF
          q2 = (pp + 2) % 4
          gwait(p)

          @pl.when(j + 2 < RPT)
          def _():
            idxload(j + 2, q2)

          scatter(pp, p)

          @pl.when(j + 2 < RPT)
          def _():
            iwait(q2)
            gstart(q2, p)
        return carry

      lax.fori_loop(0, RPT // 4, blk, 0)

      plsc.subcore_barrier()

      pltpu.sync_copy(acc.at[pl.ds(row_base, ROWS)],
                      out_hbm.at[pl.ds(r * N + row_base, ROWS)])

      @pl.when(s == 15)
      def _():
        pltpu.sync_copy(acc.at[pl.ds(16 * ROWS, 16)],
                        out_hbm.at[pl.ds(r * N + 16 * ROWS, 16)])

      plsc.subcore_barrier()

  return k(src2, dst2, xaug3, zrows)


def _tc_body(x_ref, agg_ref, wn_ref, ws_ref, b_ref, o_ref):
  wn = wn_ref[...]
  ws = ws_ref[...]
  bias = b_ref[0, :]
  for t in range(2):
    x = x_ref[0, :, t, :]
    ag = agg_ref[t, :, 0:F]
    deg = agg_ref[t, :, F:F + 1]
    mean = ag / jnp.maximum(deg, 1.0)
    acc = jnp.dot(mean, wn, preferred_element_type=jnp.float32,
                  precision=lax.Precision.HIGHEST)
    acc += jnp.dot(x, ws, preferred_element_type=jnp.float32,
                   precision=lax.Precision.HIGHEST)
    o_ref[0, :, t, :] = jnp.maximum(acc + bias, 0.0)


def kernel(X, g, W_self, W_neigh, b):
  B, n, T, f = X.shape
  src = g[0]
  dst = g[1]

  t1 = jnp.transpose(X, (0, 2, 1, 3)).reshape(NREP * N, F)
  xaug = jnp.concatenate(
      [t1,
       jnp.ones((NREP * N, 1), jnp.float32),
       jnp.zeros((NREP * N, FP - F - 1), jnp.float32)], axis=1)
  xaug3 = jnp.concatenate(
      [xaug.reshape(NREP, N, FP), jnp.zeros((NREP, ZR, FP), jnp.float32)],
      axis=1)
  zrows = jnp.zeros((ROWS, FP), jnp.float32)

  pad = EPAD - E
  src_p = jnp.concatenate(
      [src, N + (jnp.arange(pad, dtype=jnp.int32) % ZR)])
  dst_p = jnp.concatenate(
      [dst, jnp.arange(pad, dtype=jnp.int32) % N])
  src2 = src_p.reshape(CROWS, KE)
  dst2 = dst_p.reshape(CROWS, KE)

  agg = _sc_aggregate(src2, dst2, xaug3, zrows).reshape(NREP, N, FP)

  RB = 2000
  out = pl.pallas_call(
      _tc_body,
      grid=(B, N // RB),
      in_specs=[
          pl.BlockSpec((1, RB, T, F), lambda bb, nn: (bb, nn, 0, 0)),
          pl.BlockSpec((T, RB, FP), lambda bb, nn: (bb, nn, 0)),
          pl.BlockSpec((F, F), lambda bb, nn: (0, 0)),
          pl.BlockSpec((F, F), lambda bb, nn: (0, 0)),
          pl.BlockSpec((1, F), lambda bb, nn: (0, 0)),
      ],
      out_specs=pl.BlockSpec((1, RB, T, F), lambda bb, nn: (bb, nn, 0, 0)),
      out_shape=jax.ShapeDtypeStruct((B, N, T, F), jnp.float32),
  )(X, agg, W_neigh.T, W_self.T, b.reshape(1, F))
  return out


# speedup vs baseline: 1.9065x; 1.0178x over previous
import functools

import jax
import jax.numpy as jnp
from jax import lax
from jax.experimental import pallas as pl
from jax.experimental.pallas import tpu as pltpu
from jax.experimental.pallas import tpu_sc as plsc

N = 10000
E = 160000
F = 128
FP = 144
NREP = 4
KE = 128
CROWS = 1280
EPAD = CROWS * KE
RPT = CROWS // 16
NBUF = 2
ZR = 128
ROWS = 624


def _sc_aggregate(src2, dst2, xaug3, zrows):
  mesh = plsc.VectorSubcoreMesh(core_axis_name="c", subcore_axis_name="s")

  @functools.partial(
      pl.kernel,
      mesh=mesh,
      compiler_params=pltpu.CompilerParams(use_tc_tiling_on_sc=False),
      out_type=jax.ShapeDtypeStruct((NREP * N, FP), jnp.float32),
      scratch_types=(
          [pltpu.VMEM((KE,), jnp.int32)] * 4 +
          [pltpu.VMEM((KE,), jnp.int32)] * 4 +
          [pltpu.VMEM((NBUF, KE, FP), jnp.float32),
           pltpu.VMEM_SHARED((N, FP), jnp.float32),
           pltpu.SemaphoreType.DMA((NBUF,)),
           pltpu.SemaphoreType.DMA((4,))]
      ),
  )
  def k(src2_hbm, dst2_hbm, xaug_hbm, z_hbm, out_hbm,
        si0, si1, si2, si3, di0, di1, di2, di3, rows_v, acc, gsem, isem):
    sis = [si0, si1, si2, si3]
    dis = [di0, di1, di2, di3]
    c = lax.axis_index("c")
    s = lax.axis_index("s")
    row_base = s * ROWS
    cbase = s * RPT

    def idxload(j, q):
      pltpu.async_copy(src2_hbm.at[cbase + j], sis[q], isem.at[q])
      pltpu.async_copy(dst2_hbm.at[cbase + j], dis[q], isem.at[q])

    def iwait(q):
      pltpu.make_async_copy(src2_hbm.at[0], sis[q], isem.at[q]).wait()
      pltpu.make_async_copy(dst2_hbm.at[0], dis[q], isem.at[q]).wait()

    def gwait(p):
      pltpu.make_async_copy(
          xaug_hbm.at[0].at[pl.ds(0, KE)], rows_v.at[p], gsem.at[p]).wait()

    def scatter(q, p):
      pltpu.sync_copy(rows_v.at[p], acc.at[dis[q]], add=True)

    for rr in range(NREP // 2):
      r = rr * 2 + c

      def gstart(q, p):
        pltpu.async_copy(
            xaug_hbm.at[r].at[sis[q]], rows_v.at[p], gsem.at[p])

      pltpu.sync_copy(z_hbm.at[pl.ds(0, ROWS)],
                      acc.at[pl.ds(row_base, ROWS)])

      @pl.when(s == 15)
      def _():
        pltpu.sync_copy(z_hbm.at[pl.ds(0, 16)], acc.at[pl.ds(16 * ROWS, 16)])

      plsc.subcore_barrier()

      idxload(0, 0)
      idxload(1, 1)
      iwait(0)
      gstart(0, 0)
      iwait(1)
      gstart(1, 1)

      def blk(i, carry):
        for pp in range(4):
          j = i * 4 + pp
          p = pp % NBUF
          q2 = (pp + 2) % 4
          gwait(p)

          @pl.when(j + 2 < RPT)
          def _():
            idxload(j + 2, q2)

          scatter(pp, p)

          @pl.when(j + 2 < RPT)
          def _():
            iwait(q2)
            gstart(q2, p)
        return carry

      lax.fori_loop(0, RPT // 4, blk, 0)

      plsc.subcore_barrier()

      pltpu.sync_copy(acc.at[pl.ds(row_base, ROWS)],
                      out_hbm.at[pl.ds(r * N + row_base, ROWS)])

      @pl.when(s == 15)
      def _():
        pltpu.sync_copy(acc.at[pl.ds(16 * ROWS, 16)],
                        out_hbm.at[pl.ds(r * N + 16 * ROWS, 16)])

      plsc.subcore_barrier()

  return k(src2, dst2, xaug3, zrows)


def _tc_self(x_ref, ws_ref, b_ref, o_ref):
  ws = ws_ref[...]
  bias = b_ref[0, :]
  for t in range(2):
    o_ref[0, :, t, :] = bias + jnp.dot(
        x_ref[0, :, t, :], ws, preferred_element_type=jnp.float32,
        precision=lax.Precision.HIGHEST)


def _tc_fin(s_ref, agg_ref, wn_ref, o_ref):
  wn = wn_ref[...]
  for t in range(2):
    ag = agg_ref[t, :, 0:F]
    deg = agg_ref[t, :, F:F + 1]
    mean = ag / jnp.maximum(deg, 1.0)
    acc = jnp.dot(mean, wn, preferred_element_type=jnp.float32,
                  precision=lax.Precision.HIGHEST)
    o_ref[0, :, t, :] = jnp.maximum(acc + s_ref[0, :, t, :], 0.0)


def kernel(X, g, W_self, W_neigh, b):
  B, n, T, f = X.shape
  src = g[0]
  dst = g[1]

  t1 = jnp.transpose(X, (0, 2, 1, 3)).reshape(NREP * N, F)
  xaug = jnp.concatenate(
      [t1,
       jnp.ones((NREP * N, 1), jnp.float32),
       jnp.zeros((NREP * N, FP - F - 1), jnp.float32)], axis=1)
  xaug3 = jnp.concatenate(
      [xaug.reshape(NREP, N, FP), jnp.zeros((NREP, ZR, FP), jnp.float32)],
      axis=1)
  zrows = jnp.zeros((ROWS, FP), jnp.float32)

  pad = EPAD - E
  src_p = jnp.concatenate(
      [src, N + (jnp.arange(pad, dtype=jnp.int32) % ZR)])
  dst_p = jnp.concatenate(
      [dst, jnp.arange(pad, dtype=jnp.int32) % N])
  src2 = src_p.reshape(CROWS, KE)
  dst2 = dst_p.reshape(CROWS, KE)

  agg = _sc_aggregate(src2, dst2, xaug3, zrows).reshape(NREP, N, FP)

  RB = 2000
  selfterm = pl.pallas_call(
      _tc_self,
      grid=(B, N // RB),
      in_specs=[
          pl.BlockSpec((1, RB, T, F), lambda bb, nn: (bb, nn, 0, 0)),
          pl.BlockSpec((F, F), lambda bb, nn: (0, 0)),
          pl.BlockSpec((1, F), lambda bb, nn: (0, 0)),
      ],
      out_specs=pl.BlockSpec((1, RB, T, F), lambda bb, nn: (bb, nn, 0, 0)),
      out_shape=jax.ShapeDtypeStruct((B, N, T, F), jnp.float32),
  )(X, W_self.T, b.reshape(1, F))

  out = pl.pallas_call(
      _tc_fin,
      grid=(B, N // RB),
      in_specs=[
          pl.BlockSpec((1, RB, T, F), lambda bb, nn: (bb, nn, 0, 0)),
          pl.BlockSpec((T, RB, FP), lambda bb, nn: (bb, nn, 0)),
          pl.BlockSpec((F, F), lambda bb, nn: (0, 0)),
      ],
      out_specs=pl.BlockSpec((1, RB, T, F), lambda bb, nn: (bb, nn, 0, 0)),
      out_shape=jax.ShapeDtypeStruct((B, N, T, F), jnp.float32),
  )(selfterm, agg, W_neigh.T)
  return out
